# Initial kernel scaffold; baseline (speedup 1.0000x reference)
import jax
import jax.numpy as jnp
from jax.experimental import pallas as pl


def kernel(x, seq_features_v, edge_index, edge_attr, edge_rbf, params):
    raise NotImplementedError("write your pallas kernel here")


# speedup vs baseline: 2.5681x; 2.5681x over previous
import functools

import jax
import jax.numpy as jnp
from jax import lax
from jax.experimental import pallas as pl
from jax.experimental.pallas import tpu as pltpu
from jax.experimental.pallas import tpu_sc as plsc

_N = 10000
_E = 320000
_F = 128
_H = 128
_R = 16
_NC = 2
_NS = 16
_NW = _NC * _NS
_CE = 128
_EPT = ((_E + _NW * _CE - 1) // (_NW * _CE)) * _CE
_EP = _EPT * _NW
_NROW = 10112
_ROWS_PER_TILE = _NROW // _NS


def _two_mm_relu(xin, w1, b1, w2, b2):
    n = xin.shape[0]
    blk = 2000

    def body(x_ref, w1_ref, b1_ref, w2_ref, b2_ref, o1_ref, o2_ref):
        xb = x_ref[...]
        o1_ref[...] = jnp.maximum(
            jnp.dot(xb, w1_ref[...], preferred_element_type=jnp.float32)
            + b1_ref[...], 0.0)
        o2_ref[...] = jnp.maximum(
            jnp.dot(xb, w2_ref[...], preferred_element_type=jnp.float32)
            + b2_ref[...], 0.0)

    return pl.pallas_call(
        body,
        grid=(n // blk,),
        in_specs=[
            pl.BlockSpec((blk, _H), lambda i: (i, 0)),
            pl.BlockSpec((_H, _H), lambda i: (0, 0)),
            pl.BlockSpec((1, _H), lambda i: (0, 0)),
            pl.BlockSpec((_H, _H), lambda i: (0, 0)),
            pl.BlockSpec((1, _H), lambda i: (0, 0)),
        ],
        out_specs=[pl.BlockSpec((blk, _H), lambda i: (i, 0))] * 2,
        out_shape=[jax.ShapeDtypeStruct((n, _H), jnp.float32)] * 2,
    )(xin, w1, b1.reshape(1, _H), w2, b2.reshape(1, _H))


def _erbf_proj(rbf_p, w, b):
    blk = 4096

    def body(r_ref, w_ref, b_ref, o_ref):
        o_ref[...] = (
            jnp.dot(r_ref[...], w_ref[...], preferred_element_type=jnp.float32)
            + b_ref[...])

    return pl.pallas_call(
        body,
        grid=(_EP // blk,),
        in_specs=[
            pl.BlockSpec((blk, _R), lambda i: (i, 0)),
            pl.BlockSpec((_R, _H), lambda i: (0, 0)),
            pl.BlockSpec((1, _H), lambda i: (0, 0)),
        ],
        out_specs=pl.BlockSpec((blk, _H), lambda i: (i, 0)),
        out_shape=jax.ShapeDtypeStruct((_EP, _H), jnp.float32),
    )(rbf_p, w, b.reshape(1, _H))


def _sc_mesh():
    return plsc.VectorSubcoreMesh(core_axis_name="c", subcore_axis_name="s")


def _zero_accum(buf_v, acc_sh, sid):
    zero16 = jnp.zeros((16,), jnp.float32)

    def zbody(i, _):
        for j in range(_H // 16):
            buf_v[i, pl.ds(j * 16, 16)] = zero16
        return 0
    lax.fori_loop(0, _CE, zbody, 0)
    for k in range(5):
        start = jnp.minimum(sid * _ROWS_PER_TILE + k * _CE, _NROW - _CE)
        pltpu.sync_copy(buf_v, acc_sh.at[pl.ds(start, _CE)])


def _copy_out(acc_sh, out_hbm, cid, sid):
    start = sid * _ROWS_PER_TILE
    pltpu.sync_copy(acc_sh.at[pl.ds(start, _ROWS_PER_TILE)],
                    out_hbm.at[cid, pl.ds(start, _ROWS_PER_TILE)])


@functools.partial(
    pl.kernel,
    out_type=jax.ShapeDtypeStruct((_NC, _NROW, _H), jnp.float32),
    mesh=_sc_mesh(),
    scratch_types=[
        pltpu.VMEM((_CE,), jnp.int32),
        pltpu.VMEM((_CE,), jnp.int32),
        pltpu.VMEM((_CE, _H), jnp.float32),
        pltpu.VMEM((_CE, _H), jnp.float32),
        pltpu.VMEM_SHARED((_NROW, _H), jnp.float32),
        pltpu.SemaphoreType.DMA,
    ],
)
def _sc_msg_aggr(x1_hbm, src_hbm, dst_hbm, erbf_hbm, out_hbm,
                 idx_s, idx_d, rows_v, erbf_v, acc_sh, sem):
    cid = lax.axis_index("c")
    sid = lax.axis_index("s")
    _zero_accum(rows_v, acc_sh, sid)
    plsc.subcore_barrier()

    wid = sid * _NC + cid
    base_t = wid * _EPT

    def chunk(c, _):
        base = base_t + c * _CE
        pltpu.sync_copy(src_hbm.at[pl.ds(base, _CE)], idx_s)
        pltpu.sync_copy(dst_hbm.at[pl.ds(base, _CE)], idx_d)
        pltpu.async_copy(x1_hbm.at[idx_s], rows_v, sem).wait()
        pltpu.sync_copy(erbf_hbm.at[pl.ds(base, _CE)], erbf_v)

        def mul(i, _):
            for j in range(_H // 16):
                sl = pl.ds(j * 16, 16)
                rows_v[i, sl] = rows_v[i, sl] * erbf_v[i, sl]
            return 0
        lax.fori_loop(0, _CE, mul, 0)
        pltpu.sync_copy(rows_v, acc_sh.at[idx_d], add=True)
        return 0

    lax.fori_loop(0, _EPT // _CE, chunk, 0)
    plsc.subcore_barrier()
    _copy_out(acc_sh, out_hbm, cid, sid)


@functools.partial(
    pl.kernel,
    out_type=jax.ShapeDtypeStruct((_NC, _NROW, _H), jnp.float32),
    mesh=_sc_mesh(),
    scratch_types=[
        pltpu.VMEM((_CE,), jnp.int32),
        pltpu.VMEM((_CE, _H), jnp.float32),
        pltpu.VMEM_SHARED((_NROW, _H), jnp.float32),
    ],
)
def _sc_count(dst_hbm, out_hbm, idx_d, ones_v, acc_sh):
    cid = lax.axis_index("c")
    sid = lax.axis_index("s")
    _zero_accum(ones_v, acc_sh, sid)

    one16 = jnp.ones((16,), jnp.float32)

    def obody(i, _):
        for j in range(_H // 16):
            ones_v[i, pl.ds(j * 16, 16)] = one16
        return 0
    lax.fori_loop(0, _CE, obody, 0)
    plsc.subcore_barrier()

    wid = sid * _NC + cid
    base_t = wid * _EPT

    def chunk(c, _):
        base = base_t + c * _CE
        pltpu.sync_copy(dst_hbm.at[pl.ds(base, _CE)], idx_d)
        pltpu.sync_copy(ones_v, acc_sh.at[idx_d], add=True)
        return 0

    lax.fori_loop(0, _EPT // _CE, chunk, 0)
    plsc.subcore_barrier()
    _copy_out(acc_sh, out_hbm, cid, sid)


def _post(acc, cnt2, x1, x2, d, p):
    blk = 2000

    def body(a_ref, c_ref, x1_ref, x2_ref, d_ref, wl, bl, wr, br, w2, b2,
             wf, bf, o_ref):
        s = a_ref[0] + a_ref[1]
        cnt = c_ref[0][:, :1] + c_ref[1][:, :1]
        mean = s / jnp.maximum(cnt, 1.0)
        h1 = (jnp.dot(mean, wl[...], preferred_element_type=jnp.float32)
              + bl[...]
              + jnp.dot(x1_ref[...], wr[...], preferred_element_type=jnp.float32)
              + br[...])
        h1 = jnp.maximum(
            jnp.dot(h1, w2[...], preferred_element_type=jnp.float32) + b2[...],
            0.0)
        h = h1 + x2_ref[...]
        o_ref[...] = (jnp.dot(h, wf[...], preferred_element_type=jnp.float32)
                      + bf[...] + d_ref[...])

    full = lambda i: (0, 0)
    return pl.pallas_call(
        body,
        grid=(_N // blk,),
        in_specs=[
            pl.BlockSpec((_NC, blk, _H), lambda i: (0, i, 0)),
            pl.BlockSpec((_NC, blk, _H), lambda i: (0, i, 0)),
            pl.BlockSpec((blk, _H), lambda i: (i, 0)),
            pl.BlockSpec((blk, _H), lambda i: (i, 0)),
            pl.BlockSpec((blk, _H), lambda i: (i, 0)),
            pl.BlockSpec((_H, _H), full),
            pl.BlockSpec((1, _H), full),
            pl.BlockSpec((_H, _H), full),
            pl.BlockSpec((1, _H), full),
            pl.BlockSpec((_H, _H), full),
            pl.BlockSpec((1, _H), full),
            pl.BlockSpec((_H, _H), full),
            pl.BlockSpec((1, _H), full),
        ],
        out_specs=pl.BlockSpec((blk, _H), lambda i: (i, 0)),
        out_shape=jax.ShapeDtypeStruct((_N, _H), jnp.float32),
    )(acc, cnt2, x1, x2, d,
      p['W_l'], p['b_l'].reshape(1, _H),
      p['W_r'], p['b_r'].reshape(1, _H),
      p['W_rbf2'], p['b_rbf2'].reshape(1, _H),
      p['W_f'], p['b_f'].reshape(1, _H))


def kernel(x, seq_features_v, edge_index, edge_attr, edge_rbf, params):
    del seq_features_v, edge_attr
    src = edge_index[0].astype(jnp.int32)
    dst = edge_index[1].astype(jnp.int32)
    pad = _EP - _E
    src_p = jnp.concatenate([src, jnp.zeros((pad,), jnp.int32)])
    dst_p = jnp.concatenate([dst, jnp.full((pad,), _N, jnp.int32)])
    rbf_p = jnp.concatenate(
        [edge_rbf.astype(jnp.float32), jnp.zeros((pad, _R), jnp.float32)],
        axis=0)

    d1, d = _two_mm_relu(x, params['W_v1'], params['b_v1'],
                         params['W_v2'], params['b_v2'])
    cnt2 = _sc_count(dst_p)
    for b in range(2):
        p = params['blocks'][b]
        x1, x2 = _two_mm_relu(d, p['W_x1'], p['b_x1'], p['W_x2'], p['b_x2'])
        erbf = _erbf_proj(rbf_p, p['W_rbf1'], p['b_rbf1'])
        acc = _sc_msg_aggr(x1, src_p, dst_p, erbf)
        d = _post(acc, cnt2, x1, x2, d, p)
    return jnp.concatenate([d1, d], axis=1)


# speedup vs baseline: 3.4370x; 1.3383x over previous
import functools

import jax
import jax.numpy as jnp
from jax import lax
from jax.experimental import pallas as pl
from jax.experimental.pallas import tpu as pltpu
from jax.experimental.pallas import tpu_sc as plsc

_N = 10000
_E = 320000
_H = 128
_R = 16
_NC = 2
_NS = 16
_CE = 64
_NCH = 158
_EPT = _NCH * _CE
_EP = _EPT * _NS * _NC
_NROW = 10112
_ROWS_PER_TILE = _NROW // _NS
_ZCH = -(-_ROWS_PER_TILE // _CE)


def _two_mm_relu(xin, w1, b1, w2, b2):
    n = xin.shape[0]
    blk = 2000

    def body(x_ref, w1_ref, b1_ref, w2_ref, b2_ref, o1_ref, o2_ref):
        xb = x_ref[...]
        o1_ref[...] = jnp.maximum(
            jnp.dot(xb, w1_ref[...], preferred_element_type=jnp.float32)
            + b1_ref[...], 0.0)
        o2_ref[...] = jnp.maximum(
            jnp.dot(xb, w2_ref[...], preferred_element_type=jnp.float32)
            + b2_ref[...], 0.0)

    return pl.pallas_call(
        body,
        grid=(n // blk,),
        in_specs=[
            pl.BlockSpec((blk, _H), lambda i: (i, 0)),
            pl.BlockSpec((_H, _H), lambda i: (0, 0)),
            pl.BlockSpec((1, _H), lambda i: (0, 0)),
            pl.BlockSpec((_H, _H), lambda i: (0, 0)),
            pl.BlockSpec((1, _H), lambda i: (0, 0)),
        ],
        out_specs=[pl.BlockSpec((blk, _H), lambda i: (i, 0))] * 2,
        out_shape=[jax.ShapeDtypeStruct((n, _H), jnp.float32)] * 2,
    )(xin, w1, b1.reshape(1, _H), w2, b2.reshape(1, _H))


def _erbf_proj(rbf_p, w, b):
    blk = 4096

    def body(r_ref, w_ref, b_ref, o_ref):
        o_ref[...] = (
            jnp.dot(r_ref[...], w_ref[...], preferred_element_type=jnp.float32)
            + b_ref[...])

    return pl.pallas_call(
        body,
        grid=(_EP // blk,),
        in_specs=[
            pl.BlockSpec((blk, _R), lambda i: (i, 0)),
            pl.BlockSpec((_R, _H), lambda i: (0, 0)),
            pl.BlockSpec((1, _H), lambda i: (0, 0)),
        ],
        out_specs=pl.BlockSpec((blk, _H), lambda i: (i, 0)),
        out_shape=jax.ShapeDtypeStruct((_EP, _H), jnp.float32),
    )(rbf_p, w, b.reshape(1, _H))


def _sc_mesh():
    return plsc.VectorSubcoreMesh(core_axis_name="c", subcore_axis_name="s")


def _fill(buf_v, w, val16):
    @plsc.parallel_loop(0, _CE, step=1, unroll=4)
    def _(i):
        for j in range(w // 16):
            buf_v[i, pl.ds(j * 16, 16)] = val16


def _zero_accum(zbuf, acc_sh, sid):
    for k in range(_ZCH):
        start = jnp.minimum(sid * _ROWS_PER_TILE + k * _CE, _NROW - _CE)
        pltpu.sync_copy(zbuf, acc_sh.at[pl.ds(start, _CE)])


@functools.partial(
    pl.kernel,
    out_type=jax.ShapeDtypeStruct((_NC, _NROW, _H), jnp.float32),
    mesh=_sc_mesh(),
    scratch_types=[
        pltpu.VMEM((_EPT,), jnp.int32),
        pltpu.VMEM((2, _CE), jnp.int32),
        pltpu.VMEM((2, _CE, _H), jnp.float32),
        pltpu.VMEM((2, _CE, _H), jnp.float32),
        pltpu.VMEM_SHARED((_NROW, _H), jnp.float32),
        pltpu.SemaphoreType.DMA,
        pltpu.SemaphoreType.DMA,
        pltpu.SemaphoreType.DMA,
        pltpu.SemaphoreType.DMA,
    ],
)
def _sc_msg_aggr(x1_hbm, src_hbm, dst_hbm, erbf_hbm, out_hbm,
                 idx_s, idx_d, rows, erbf, acc_sh,
                 gsem0, gsem1, ssem0, ssem1):
    cid = lax.axis_index("c")
    sid = lax.axis_index("s")
    gsem = (gsem0, gsem1)
    ssem = (ssem0, ssem1)

    zero16 = jnp.zeros((16,), jnp.float32)
    _fill(rows.at[0], _H, zero16)
    _zero_accum(rows.at[0], acc_sh, sid)
    plsc.subcore_barrier()

    base_t = (cid * _NS + sid) * _EPT
    pltpu.sync_copy(src_hbm.at[pl.ds(base_t, _EPT)], idx_s)

    def start(cc, b):
        pltpu.async_copy(dst_hbm.at[pl.ds(base_t + cc * _CE, _CE)],
                         idx_d.at[b], gsem[b])
        pltpu.async_copy(
            x1_hbm.at[idx_s.at[pl.ds(cc * _CE, _CE)]], rows.at[b], gsem[b])
        pltpu.async_copy(erbf_hbm.at[pl.ds(base_t + cc * _CE, _CE)],
                         erbf.at[b], gsem[b])

    def wait_loads(cc, b):
        pltpu.make_async_copy(dst_hbm.at[pl.ds(base_t + cc * _CE, _CE)],
                              idx_d.at[b], gsem[b]).wait()
        pltpu.make_async_copy(
            x1_hbm.at[idx_s.at[pl.ds(cc * _CE, _CE)]], rows.at[b],
            gsem[b]).wait()
        pltpu.make_async_copy(erbf_hbm.at[pl.ds(base_t + cc * _CE, _CE)],
                              erbf.at[b], gsem[b]).wait()

    def start_scatter(b):
        pltpu.async_copy(rows.at[b], acc_sh.at[idx_d.at[b]], ssem[b],
                         add=True)

    def wait_scatter(b):
        pltpu.make_async_copy(rows.at[b], acc_sh.at[idx_d.at[b]],
                              ssem[b]).wait()

    def mul(b):
        rb = rows.at[b]
        eb = erbf.at[b]

        @plsc.parallel_loop(0, _CE, step=1, unroll=4)
        def _(i):
            for j in range(_H // 16):
                sl = pl.ds(j * 16, 16)
                rb[i, sl] = rb[i, sl] * eb[i, sl]

    start(0, 0)

    @pl.loop(0, _NCH, step=2)
    def _(c):
        for b in range(2):
            cc = c + b
            nb = 1 - b
            wait_loads(cc, b)

            @pl.when(cc > 0)
            def _():
                wait_scatter(nb)

            @pl.when(cc < _NCH - 1)
            def _():
                start(cc + 1, nb)

            mul(b)
            start_scatter(b)

    wait_scatter(1)
    plsc.subcore_barrier()
    start_r = sid * _ROWS_PER_TILE
    pltpu.sync_copy(acc_sh.at[pl.ds(start_r, _ROWS_PER_TILE)],
                    out_hbm.at[cid, pl.ds(start_r, _ROWS_PER_TILE)])


@functools.partial(
    pl.kernel,
    out_type=jax.ShapeDtypeStruct((_NC, _NROW, _H), jnp.float32),
    mesh=_sc_mesh(),
    scratch_types=[
        pltpu.VMEM((_EPT,), jnp.int32),
        pltpu.VMEM((_CE, _H), jnp.float32),
        pltpu.VMEM_SHARED((_NROW, _H), jnp.float32),
        pltpu.SemaphoreType.DMA,
    ],
)
def _sc_count(dst_hbm, out_hbm, idx_d, ones_v, acc_sh, csem):
    cid = lax.axis_index("c")
    sid = lax.axis_index("s")
    _fill(ones_v, _H, jnp.zeros((16,), jnp.float32))
    _zero_accum(ones_v, acc_sh, sid)
    _fill(ones_v, _H, jnp.ones((16,), jnp.float32))
    plsc.subcore_barrier()

    base_t = (cid * _NS + sid) * _EPT
    pltpu.sync_copy(dst_hbm.at[pl.ds(base_t, _EPT)], idx_d)

    @pl.loop(0, _NCH)
    def _(c):
        dsl = idx_d.at[pl.ds(c * _CE, _CE)]
        pltpu.async_copy(ones_v, acc_sh.at[dsl], csem, add=True)
        pltpu.make_async_copy(ones_v, acc_sh.at[dsl], csem).wait()

    plsc.subcore_barrier()

    start = sid * _ROWS_PER_TILE
    pltpu.sync_copy(acc_sh.at[pl.ds(start, _ROWS_PER_TILE)],
                    out_hbm.at[cid, pl.ds(start, _ROWS_PER_TILE)])


def _post(acc, cnt2, x1, x2, d, p):
    blk = 2000

    def body(a_ref, c_ref, x1_ref, x2_ref, d_ref, wl, bl, wr, br, w2, b2,
             wf, bf, o_ref):
        cnt = jnp.maximum(c_ref[0][:, :1] + c_ref[1][:, :1], 1.0)
        mean = (a_ref[0] + a_ref[1]) / cnt
        h1 = (jnp.dot(mean, wl[...], preferred_element_type=jnp.float32)
              + bl[...]
              + jnp.dot(x1_ref[...], wr[...],
                        preferred_element_type=jnp.float32)
              + br[...])
        h1 = jnp.maximum(
            jnp.dot(h1, w2[...], preferred_element_type=jnp.float32) + b2[...],
            0.0)
        h = h1 + x2_ref[...]
        o_ref[...] = (jnp.dot(h, wf[...], preferred_element_type=jnp.float32)
                      + bf[...] + d_ref[...])

    full = lambda i: (0, 0)
    return pl.pallas_call(
        body,
        grid=(_N // blk,),
        in_specs=[
            pl.BlockSpec((_NC, blk, _H), lambda i: (0, i, 0)),
            pl.BlockSpec((_NC, blk, _H), lambda i: (0, i, 0)),
            pl.BlockSpec((blk, _H), lambda i: (i, 0)),
            pl.BlockSpec((blk, _H), lambda i: (i, 0)),
            pl.BlockSpec((blk, _H), lambda i: (i, 0)),
            pl.BlockSpec((_H, _H), full),
            pl.BlockSpec((1, _H), full),
            pl.BlockSpec((_H, _H), full),
            pl.BlockSpec((1, _H), full),
            pl.BlockSpec((_H, _H), full),
            pl.BlockSpec((1, _H), full),
            pl.BlockSpec((_H, _H), full),
            pl.BlockSpec((1, _H), full),
        ],
        out_specs=pl.BlockSpec((blk, _H), lambda i: (i, 0)),
        out_shape=jax.ShapeDtypeStruct((_N, _H), jnp.float32),
    )(acc, cnt2, x1, x2, d,
      p['W_l'], p['b_l'].reshape(1, _H),
      p['W_r'], p['b_r'].reshape(1, _H),
      p['W_rbf2'], p['b_rbf2'].reshape(1, _H),
      p['W_f'], p['b_f'].reshape(1, _H))


def kernel(x, seq_features_v, edge_index, edge_attr, edge_rbf, params):
    del seq_features_v, edge_attr
    src = edge_index[0].astype(jnp.int32)
    dst = edge_index[1].astype(jnp.int32)
    pad = _EP - _E
    src_p = jnp.concatenate([src, jnp.zeros((pad,), jnp.int32)])
    dst_p = jnp.concatenate([dst, jnp.full((pad,), _N, jnp.int32)])
    rbf_p = jnp.concatenate(
        [edge_rbf.astype(jnp.float32), jnp.zeros((pad, _R), jnp.float32)],
        axis=0)

    d1, d = _two_mm_relu(x, params['W_v1'], params['b_v1'],
                         params['W_v2'], params['b_v2'])
    cnt2 = _sc_count(dst_p)
    for b in range(2):
        p = params['blocks'][b]
        x1, x2 = _two_mm_relu(d, p['W_x1'], p['b_x1'],
                              p['W_x2'], p['b_x2'])
        erbf = _erbf_proj(rbf_p, p['W_rbf1'], p['b_rbf1'])
        acc = _sc_msg_aggr(x1, src_p, dst_p, erbf)
        d = _post(acc, cnt2, x1, x2, d, p)
    return jnp.concatenate([d1, d], axis=1)


# speedup vs baseline: 3.4404x; 1.0010x over previous
import functools

import jax
import jax.numpy as jnp
from jax import lax
from jax.experimental import pallas as pl
from jax.experimental.pallas import tpu as pltpu
from jax.experimental.pallas import tpu_sc as plsc

_N = 10000
_E = 320000
_H = 128
_R = 16
_NC = 2
_NS = 16
_CE = 64
_NCH = 158
_EPT = _NCH * _CE
_EP = _EPT * _NS * _NC
_NROW = 10112
_ROWS_PER_TILE = _NROW // _NS
_ZCH = -(-_ROWS_PER_TILE // _CE)


def _two_mm_relu(xin, w1, b1, w2, b2):
    n = xin.shape[0]
    blk = 2000

    def body(x_ref, w1_ref, b1_ref, w2_ref, b2_ref, o1_ref, o2_ref):
        xb = x_ref[...]
        o1_ref[...] = jnp.maximum(
            jnp.dot(xb, w1_ref[...], preferred_element_type=jnp.float32)
            + b1_ref[...], 0.0)
        o2_ref[...] = jnp.maximum(
            jnp.dot(xb, w2_ref[...], preferred_element_type=jnp.float32)
            + b2_ref[...], 0.0)

    return pl.pallas_call(
        body,
        grid=(n // blk,),
        in_specs=[
            pl.BlockSpec((blk, _H), lambda i: (i, 0)),
            pl.BlockSpec((_H, _H), lambda i: (0, 0)),
            pl.BlockSpec((1, _H), lambda i: (0, 0)),
            pl.BlockSpec((_H, _H), lambda i: (0, 0)),
            pl.BlockSpec((1, _H), lambda i: (0, 0)),
        ],
        out_specs=[pl.BlockSpec((blk, _H), lambda i: (i, 0))] * 2,
        out_shape=[jax.ShapeDtypeStruct((n, _H), jnp.float32)] * 2,
    )(xin, w1, b1.reshape(1, _H), w2, b2.reshape(1, _H))


def _erbf_proj(rbf_p, w, b):
    blk = 4096

    def body(r_ref, w_ref, b_ref, o_ref):
        o_ref[...] = (
            jnp.dot(r_ref[...], w_ref[...], preferred_element_type=jnp.float32)
            + b_ref[...])

    return pl.pallas_call(
        body,
        grid=(_EP // blk,),
        in_specs=[
            pl.BlockSpec((blk, _R), lambda i: (i, 0)),
            pl.BlockSpec((_R, _H), lambda i: (0, 0)),
            pl.BlockSpec((1, _H), lambda i: (0, 0)),
        ],
        out_specs=pl.BlockSpec((blk, _H), lambda i: (i, 0)),
        out_shape=jax.ShapeDtypeStruct((_EP, _H), jnp.float32),
    )(rbf_p, w, b.reshape(1, _H))


def _sc_mesh():
    return plsc.VectorSubcoreMesh(core_axis_name="c", subcore_axis_name="s")


def _fill(buf_v, w, val16):
    @plsc.parallel_loop(0, _CE, step=1, unroll=4)
    def _(i):
        for j in range(w // 16):
            buf_v[i, pl.ds(j * 16, 16)] = val16


def _zero_accum(zbuf, acc_sh, sid):
    for k in range(_ZCH):
        start = jnp.minimum(sid * _ROWS_PER_TILE + k * _CE, _NROW - _CE)
        pltpu.sync_copy(zbuf, acc_sh.at[pl.ds(start, _CE)])


@functools.partial(
    pl.kernel,
    out_type=jax.ShapeDtypeStruct((_NC, _NROW, _H), jnp.float32),
    mesh=_sc_mesh(),
    scratch_types=[
        pltpu.VMEM((_EPT,), jnp.int32),
        pltpu.VMEM((2, _CE), jnp.int32),
        pltpu.VMEM((2, _CE, _H), jnp.float32),
        pltpu.VMEM((2, _CE, _H), jnp.float32),
        pltpu.VMEM_SHARED((_NROW, _H), jnp.float32),
        pltpu.SemaphoreType.DMA,
        pltpu.SemaphoreType.DMA,
        pltpu.SemaphoreType.DMA,
        pltpu.SemaphoreType.DMA,
    ],
)
def _sc_msg_aggr(x1_hbm, src_hbm, dst_hbm, erbf_hbm, out_hbm,
                 idx_s, idx_d, rows, erbf, acc_sh,
                 gsem0, gsem1, ssem0, ssem1):
    cid = lax.axis_index("c")
    sid = lax.axis_index("s")
    gsem = (gsem0, gsem1)
    ssem = (ssem0, ssem1)

    zero16 = jnp.zeros((16,), jnp.float32)
    _fill(rows.at[0], _H, zero16)
    _zero_accum(rows.at[0], acc_sh, sid)
    plsc.subcore_barrier()

    base_t = (cid * _NS + sid) * _EPT
    pltpu.sync_copy(src_hbm.at[pl.ds(base_t, _EPT)], idx_s)

    def start(cc, b):
        pltpu.async_copy(dst_hbm.at[pl.ds(base_t + cc * _CE, _CE)],
                         idx_d.at[b], gsem[b])
        pltpu.async_copy(
            x1_hbm.at[idx_s.at[pl.ds(cc * _CE, _CE)]], rows.at[b], gsem[b])
        pltpu.async_copy(erbf_hbm.at[pl.ds(base_t + cc * _CE, _CE)],
                         erbf.at[b], gsem[b])

    def wait_loads(cc, b):
        pltpu.make_async_copy(dst_hbm.at[pl.ds(base_t + cc * _CE, _CE)],
                              idx_d.at[b], gsem[b]).wait()
        pltpu.make_async_copy(
            x1_hbm.at[idx_s.at[pl.ds(cc * _CE, _CE)]], rows.at[b],
            gsem[b]).wait()
        pltpu.make_async_copy(erbf_hbm.at[pl.ds(base_t + cc * _CE, _CE)],
                              erbf.at[b], gsem[b]).wait()

    def start_scatter(b):
        pltpu.async_copy(rows.at[b], acc_sh.at[idx_d.at[b]], ssem[b],
                         add=True)

    def wait_scatter(b):
        pltpu.make_async_copy(rows.at[b], acc_sh.at[idx_d.at[b]],
                              ssem[b]).wait()

    def mul(b):
        rb = rows.at[b]
        eb = erbf.at[b]

        @plsc.parallel_loop(0, _CE, step=1, unroll=4)
        def _(i):
            for j in range(_H // 16):
                sl = pl.ds(j * 16, 16)
                rb[i, sl] = rb[i, sl] * eb[i, sl]

    start(0, 0)

    @pl.loop(0, _NCH, step=2)
    def _(c):
        for b in range(2):
            cc = c + b
            nb = 1 - b
            wait_loads(cc, b)

            @pl.when(cc > 0)
            def _():
                wait_scatter(nb)

            @pl.when(cc < _NCH - 1)
            def _():
                start(cc + 1, nb)

            mul(b)
            start_scatter(b)

    wait_scatter(1)
    plsc.subcore_barrier()
    start_r = sid * _ROWS_PER_TILE
    pltpu.sync_copy(acc_sh.at[pl.ds(start_r, _ROWS_PER_TILE)],
                    out_hbm.at[cid, pl.ds(start_r, _ROWS_PER_TILE)])


@functools.partial(
    pl.kernel,
    out_type=jax.ShapeDtypeStruct((_NC, _NROW, _H), jnp.float32),
    mesh=_sc_mesh(),
    scratch_types=[
        pltpu.VMEM((_EPT,), jnp.int32),
        pltpu.VMEM((_CE, _H), jnp.float32),
        pltpu.VMEM_SHARED((_NROW, _H), jnp.float32),
        pltpu.SemaphoreType.DMA,
    ],
)
def _sc_count(dst_hbm, out_hbm, idx_d, ones_v, acc_sh, csem):
    cid = lax.axis_index("c")
    sid = lax.axis_index("s")
    _fill(ones_v, _H, jnp.zeros((16,), jnp.float32))
    _zero_accum(ones_v, acc_sh, sid)
    _fill(ones_v, _H, jnp.ones((16,), jnp.float32))
    plsc.subcore_barrier()

    base_t = (cid * _NS + sid) * _EPT
    pltpu.sync_copy(dst_hbm.at[pl.ds(base_t, _EPT)], idx_d)

    @pl.loop(0, _NCH)
    def _(c):
        dsl = idx_d.at[pl.ds(c * _CE, _CE)]
        pltpu.async_copy(ones_v, acc_sh.at[dsl], csem, add=True)
        pltpu.make_async_copy(ones_v, acc_sh.at[dsl], csem).wait()

    plsc.subcore_barrier()

    start = sid * _ROWS_PER_TILE
    pltpu.sync_copy(acc_sh.at[pl.ds(start, _ROWS_PER_TILE)],
                    out_hbm.at[cid, pl.ds(start, _ROWS_PER_TILE)])


def _post(acc, cnt2, x1, x2, d, p):
    blk = 2000

    def body(a_ref, c_ref, x1_ref, x2_ref, d_ref, wl, bl, wr, br, w2, b2,
             wf, bf, o_ref):
        cnt = jnp.maximum(c_ref[0][:, :1] + c_ref[1][:, :1], 1.0)
        mean = (a_ref[0] + a_ref[1]) / cnt
        h1 = (jnp.dot(mean, wl[...], preferred_element_type=jnp.float32)
              + bl[...]
              + jnp.dot(x1_ref[...], wr[...],
                        preferred_element_type=jnp.float32)
              + br[...])
        h1 = jnp.maximum(
            jnp.dot(h1, w2[...], preferred_element_type=jnp.float32) + b2[...],
            0.0)
        h = h1 + x2_ref[...]
        o_ref[...] = (jnp.dot(h, wf[...], preferred_element_type=jnp.float32)
                      + bf[...] + d_ref[...])

    full = lambda i: (0, 0)
    return pl.pallas_call(
        body,
        grid=(_N // blk,),
        in_specs=[
            pl.BlockSpec((_NC, blk, _H), lambda i: (0, i, 0)),
            pl.BlockSpec((_NC, blk, _H), lambda i: (0, i, 0)),
            pl.BlockSpec((blk, _H), lambda i: (i, 0)),
            pl.BlockSpec((blk, _H), lambda i: (i, 0)),
            pl.BlockSpec((blk, _H), lambda i: (i, 0)),
            pl.BlockSpec((_H, _H), full),
            pl.BlockSpec((1, _H), full),
            pl.BlockSpec((_H, _H), full),
            pl.BlockSpec((1, _H), full),
            pl.BlockSpec((_H, _H), full),
            pl.BlockSpec((1, _H), full),
            pl.BlockSpec((_H, _H), full),
            pl.BlockSpec((1, _H), full),
        ],
        out_specs=pl.BlockSpec((blk, _H), lambda i: (i, 0)),
        out_shape=jax.ShapeDtypeStruct((_N, _H), jnp.float32),
    )(acc, cnt2, x1, x2, d,
      p['W_l'], p['b_l'].reshape(1, _H),
      p['W_r'], p['b_r'].reshape(1, _H),
      p['W_rbf2'], p['b_rbf2'].reshape(1, _H),
      p['W_f'], p['b_f'].reshape(1, _H))


def kernel(x, seq_features_v, edge_index, edge_attr, edge_rbf, params):
    del seq_features_v, edge_attr
    src = edge_index[0].astype(jnp.int32)
    dst = edge_index[1].astype(jnp.int32)
    pad = _EP - _E
    src_p = jnp.concatenate([src, jnp.zeros((pad,), jnp.int32)])
    dst_p = jnp.concatenate([dst, jnp.full((pad,), _N, jnp.int32)])
    rbf_p = jnp.concatenate(
        [edge_rbf.astype(jnp.float32), jnp.zeros((pad, _R), jnp.float32)],
        axis=0)

    d1, d = _two_mm_relu(x, params['W_v1'], params['b_v1'],
                         params['W_v2'], params['b_v2'])
    cnt2 = _sc_count(dst_p)
    erbfs = [_erbf_proj(rbf_p, params['blocks'][b]['W_rbf1'],
                        params['blocks'][b]['b_rbf1']) for b in range(2)]
    for b in range(2):
        p = params['blocks'][b]
        x1, x2 = _two_mm_relu(d, p['W_x1'], p['b_x1'],
                              p['W_x2'], p['b_x2'])
        acc = _sc_msg_aggr(x1, src_p, dst_p, erbfs[b])
        d = _post(acc, cnt2, x1, x2, d, p)
    return jnp.concatenate([d1, d], axis=1)
